# baseline (speedup 1.0000x reference)
import functools

import jax
import jax.numpy as jnp
from jax import lax
from jax.experimental import pallas as pl
from jax.experimental.pallas import tpu as pltpu
from jax.experimental.pallas import tpu_sc as plsc

H = 1024
NW = 32
CHUNK = 16


NBUF = 6
LAG = NBUF // 2


def _sc_gather(ids_flat, word_emb, n_tokens):
    tok_per_w = n_tokens // NW
    nchunk = tok_per_w // CHUNK
    mesh = plsc.VectorSubcoreMesh(core_axis_name="c", subcore_axis_name="s")

    @functools.partial(
        pl.kernel,
        out_type=jax.ShapeDtypeStruct((n_tokens, H), jnp.float32),
        mesh=mesh,
        scratch_types=(
            [pltpu.VMEM((tok_per_w,), jnp.int32)]
            + [pltpu.VMEM((CHUNK, H), jnp.float32) for _ in range(NBUF)]
            + [pltpu.SemaphoreType.DMA for _ in range(2 * LAG)]
        ),
    )
    def gather_kernel(ids_hbm, word_hbm, out_hbm, idx_v, *bufs_and_sems):
        bufs = bufs_and_sems[:NBUF]
        gsems = bufs_and_sems[NBUF:NBUF + LAG]
        osems = bufs_and_sems[NBUF + LAG:]
        wid = lax.axis_index("s") * 2 + lax.axis_index("c")
        base = wid * tok_per_w
        pltpu.sync_copy(ids_hbm.at[pl.ds(base, tok_per_w)], idx_v)

        def start_gather(c):
            return pltpu.async_copy(
                word_hbm.at[idx_v.at[pl.ds(c * CHUNK, CHUNK)]],
                bufs[c % NBUF], gsems[c % LAG])

        def start_out(c):
            return pltpu.async_copy(
                bufs[c % NBUF], out_hbm.at[pl.ds(base + c * CHUNK, CHUNK)],
                osems[c % LAG])

        gathers = [None] * nchunk
        outs = [None] * nchunk
        for c in range(min(LAG, nchunk)):
            gathers[c] = start_gather(c)
        for c in range(nchunk):
            gathers[c].wait()
            if c >= LAG:
                outs[c - LAG].wait()
            if c + LAG < nchunk:
                gathers[c + LAG] = start_gather(c + LAG)
            outs[c] = start_out(c)
        for c in range(max(0, nchunk - LAG), nchunk):
            outs[c].wait()

    return gather_kernel(ids_flat, word_emb)


def _tc_add_ln(gathered, tt_col, pos_emb, type_emb_pad, gamma2, beta2,
               batch, seq_len, blk):
    s_blocks = seq_len // blk
    n_tokens = batch * seq_len

    def body(g_ref, tt_ref, p_ref, te_ref, ga_ref, be_ref, o_ref):
        t = tt_ref[...].astype(jnp.float32)
        e0 = te_ref[0:1, :]
        e1 = te_ref[1:2, :]
        x = g_ref[...] + p_ref[...] + e0 + t * (e1 - e0)
        mean = jnp.mean(x, axis=-1, keepdims=True)
        var = jnp.mean(x * x, axis=-1, keepdims=True) - mean * mean
        inv = lax.rsqrt(var + 1e-12)
        o_ref[...] = (x - mean) * (inv * ga_ref[...]) + be_ref[...]

    return pl.pallas_call(
        body,
        grid=(s_blocks, batch),
        in_specs=[
            pl.BlockSpec((blk, H), lambda j, b: (b * s_blocks + j, 0)),
            pl.BlockSpec((blk, 1), lambda j, b: (b * s_blocks + j, 0)),
            pl.BlockSpec((blk, H), lambda j, b: (j, 0)),
            pl.BlockSpec((8, H), lambda j, b: (0, 0)),
            pl.BlockSpec((1, H), lambda j, b: (0, 0)),
            pl.BlockSpec((1, H), lambda j, b: (0, 0)),
        ],
        out_specs=pl.BlockSpec((blk, H), lambda j, b: (b * s_blocks + j, 0)),
        out_shape=jax.ShapeDtypeStruct((n_tokens, H), jnp.float32),
    )(gathered, tt_col, pos_emb, type_emb_pad, gamma2, beta2)


def kernel(input_ids, token_type_ids, word_emb, pos_emb, type_emb,
           ln_gamma, ln_beta):
    b, s = input_ids.shape
    n_tokens = b * s
    ids_flat = input_ids.reshape(n_tokens)
    gathered = _sc_gather(ids_flat, word_emb, n_tokens)

    tt_col = token_type_ids.reshape(n_tokens, 1).astype(jnp.int8)
    type_emb_pad = jnp.concatenate(
        [type_emb, jnp.zeros((6, H), jnp.float32)], axis=0)
    gamma2 = ln_gamma.reshape(1, H)
    beta2 = ln_beta.reshape(1, H)

    out = _tc_add_ln(gathered, tt_col, pos_emb, type_emb_pad, gamma2, beta2,
                     b, s, blk=2048)
    return out.reshape(b, s, H)


# speedup vs baseline: 1.0049x; 1.0049x over previous
import functools

import jax
import jax.numpy as jnp
from jax import lax
from jax.experimental import pallas as pl
from jax.experimental.pallas import tpu as pltpu
from jax.experimental.pallas import tpu_sc as plsc

H = 1024
NW = 32
CHUNK = 32


def _sc_gather(ids_flat, word_emb, n_tokens):
    tok_per_w = n_tokens // NW
    nchunk = tok_per_w // CHUNK
    mesh = plsc.VectorSubcoreMesh(core_axis_name="c", subcore_axis_name="s")

    @functools.partial(
        pl.kernel,
        out_type=jax.ShapeDtypeStruct((n_tokens, H), jnp.float32),
        mesh=mesh,
        scratch_types=[
            pltpu.VMEM((tok_per_w,), jnp.int32),
            pltpu.VMEM((CHUNK, H), jnp.float32),
            pltpu.VMEM((CHUNK, H), jnp.float32),
            pltpu.VMEM((CHUNK, H), jnp.float32),
            pltpu.SemaphoreType.DMA,
            pltpu.SemaphoreType.DMA,
            pltpu.SemaphoreType.DMA,
            pltpu.SemaphoreType.DMA,
        ],
    )
    def gather_kernel(ids_hbm, word_hbm, out_hbm, idx_v, rows0, rows1, rows2,
                      gsem0, gsem1, osem0, osem1):
        wid = lax.axis_index("s") * 2 + lax.axis_index("c")
        base = wid * tok_per_w
        pltpu.sync_copy(ids_hbm.at[pl.ds(base, tok_per_w)], idx_v)
        bufs = (rows0, rows1, rows2)
        gsems = (gsem0, gsem1)
        osems = (osem0, osem1)

        def start_gather(c):
            return pltpu.async_copy(
                word_hbm.at[idx_v.at[pl.ds(c * CHUNK, CHUNK)]],
                bufs[c % 3], gsems[c % 2])

        def start_out(c):
            return pltpu.async_copy(
                bufs[c % 3], out_hbm.at[pl.ds(base + c * CHUNK, CHUNK)],
                osems[c % 2])

        gathers = [None] * nchunk
        outs = [None] * nchunk
        gathers[0] = start_gather(0)
        if nchunk > 1:
            gathers[1] = start_gather(1)
        for c in range(nchunk):
            gathers[c].wait()
            if c >= 1:
                outs[c - 1].wait()
            if c + 2 < nchunk:
                gathers[c + 2] = start_gather(c + 2)
            outs[c] = start_out(c)
        outs[nchunk - 1].wait()

    return gather_kernel(ids_flat, word_emb)


def _tc_add_ln(gathered, tt_col, pos_emb, type_emb_pad, gamma2, beta2,
               batch, seq_len, blk):
    s_blocks = seq_len // blk
    n_tokens = batch * seq_len

    def body(g_ref, tt_ref, p_ref, te_ref, ga_ref, be_ref, o_ref):
        t = tt_ref[...].astype(jnp.float32)
        e0 = te_ref[0:1, :]
        e1 = te_ref[1:2, :]
        x = g_ref[...] + p_ref[...] + e0 + t * (e1 - e0)
        mean = jnp.mean(x, axis=-1, keepdims=True)
        var = jnp.mean(x * x, axis=-1, keepdims=True) - mean * mean
        inv = lax.rsqrt(var + 1e-12)
        o_ref[...] = (x - mean) * (inv * ga_ref[...]) + be_ref[...]

    return pl.pallas_call(
        body,
        grid=(s_blocks, batch),
        in_specs=[
            pl.BlockSpec((blk, H), lambda j, b: (b * s_blocks + j, 0)),
            pl.BlockSpec((blk, 1), lambda j, b: (b * s_blocks + j, 0)),
            pl.BlockSpec((blk, H), lambda j, b: (j, 0)),
            pl.BlockSpec((8, H), lambda j, b: (0, 0)),
            pl.BlockSpec((1, H), lambda j, b: (0, 0)),
            pl.BlockSpec((1, H), lambda j, b: (0, 0)),
        ],
        out_specs=pl.BlockSpec((blk, H), lambda j, b: (b * s_blocks + j, 0)),
        out_shape=jax.ShapeDtypeStruct((n_tokens, H), jnp.float32),
    )(gathered, tt_col, pos_emb, type_emb_pad, gamma2, beta2)


def kernel(input_ids, token_type_ids, word_emb, pos_emb, type_emb,
           ln_gamma, ln_beta):
    b, s = input_ids.shape
    n_tokens = b * s
    ids_flat = input_ids.reshape(n_tokens)
    gathered = _sc_gather(ids_flat, word_emb, n_tokens)

    tt_col = token_type_ids.reshape(n_tokens, 1).astype(jnp.int8)
    type_emb_pad = jnp.concatenate(
        [type_emb, jnp.zeros((6, H), jnp.float32)], axis=0)
    gamma2 = ln_gamma.reshape(1, H)
    beta2 = ln_beta.reshape(1, H)

    out = _tc_add_ln(gathered, tt_col, pos_emb, type_emb_pad, gamma2, beta2,
                     b, s, blk=2048)
    return out.reshape(b, s, H)
